# Initial kernel scaffold; baseline (speedup 1.0000x reference)
import jax
import jax.numpy as jnp
from jax.experimental import pallas as pl


def kernel(inputs, table):
    raise NotImplementedError("write your pallas kernel here")


# speedup vs baseline: 1.7969x; 1.7969x over previous
import functools

import jax
import jax.numpy as jnp
from jax import lax
from jax.experimental import pallas as pl
from jax.experimental.pallas import tpu as pltpu
from jax.experimental.pallas import tpu_sc as plsc

DIM = 300
CHUNK = 128


@functools.lru_cache(maxsize=None)
def _make_gather(B, b_per_w, n_chunks, NC):
    mesh = plsc.VectorSubcoreMesh(core_axis_name="c", subcore_axis_name="s")

    @functools.partial(
        pl.kernel,
        mesh=mesh,
        out_type=jax.ShapeDtypeStruct((B, 384), jnp.float32),
        scratch_types=[
            pltpu.VMEM((CHUNK,), jnp.int32),
            pltpu.VMEM((CHUNK, 128), jnp.float32),
            pltpu.VMEM((CHUNK, 128), jnp.float32),
            pltpu.VMEM((CHUNK, 128), jnp.float32),
            pltpu.SemaphoreType.DMA,
        ],
    )
    def k(table_hbm, tail_hbm, idx_hbm, out_hbm, idx_v, buf0, buf1, buf2, sem):
        wid = lax.axis_index("s") * NC + lax.axis_index("c")
        base = wid * b_per_w

        def body(c, carry):
            off = base + c * CHUNK
            pltpu.sync_copy(idx_hbm.at[pl.ds(off, CHUNK)], idx_v)
            pltpu.async_copy(table_hbm.at[idx_v, pl.ds(0, 128)], buf0, sem).wait()
            pltpu.sync_copy(buf0, out_hbm.at[pl.ds(off, CHUNK), pl.ds(0, 128)])
            pltpu.async_copy(table_hbm.at[idx_v, pl.ds(128, 128)], buf1, sem).wait()
            pltpu.sync_copy(buf1, out_hbm.at[pl.ds(off, CHUNK), pl.ds(128, 128)])
            pltpu.async_copy(tail_hbm.at[idx_v], buf2, sem).wait()
            pltpu.sync_copy(buf2, out_hbm.at[pl.ds(off, CHUNK), pl.ds(256, 128)])
            return carry

        lax.fori_loop(0, n_chunks, body, 0)

    return k


def kernel(inputs, table):
    bsz, seq = inputs.shape
    B = bsz * seq
    idx = inputs.reshape(B).astype(jnp.int32)
    tail = jnp.pad(table[:, 256:], ((0, 0), (0, 84)))
    info = plsc.get_sparse_core_info()
    NC, NS = info.num_cores, info.num_subcores
    NW = NC * NS
    b_per_w = B // NW
    n_chunks = b_per_w // CHUNK
    out = _make_gather(B, b_per_w, n_chunks, NC)(table, tail, idx)
    return out[:, :DIM].reshape(bsz, seq, DIM)


# speedup vs baseline: 2.0304x; 1.1300x over previous
import functools

import jax
import jax.numpy as jnp
from jax import lax
from jax.experimental import pallas as pl
from jax.experimental.pallas import tpu as pltpu
from jax.experimental.pallas import tpu_sc as plsc

DIM = 300
CHUNK = 128
NSLOT = 6


@functools.lru_cache(maxsize=None)
def _make_gather(B, b_per_w, n_chunks, NC):
    assert n_chunks % 2 == 0
    mesh = plsc.VectorSubcoreMesh(core_axis_name="c", subcore_axis_name="s")

    @functools.partial(
        pl.kernel,
        mesh=mesh,
        out_type=jax.ShapeDtypeStruct((B, 384), jnp.float32),
        scratch_types=[
            pltpu.VMEM((b_per_w,), jnp.int32),
            *([pltpu.VMEM((CHUNK, 128), jnp.float32)] * NSLOT),
            *([pltpu.SemaphoreType.DMA] * NSLOT),
            *([pltpu.SemaphoreType.DMA] * NSLOT),
        ],
    )
    def k(table_hbm, tail_hbm, idx_hbm, out_hbm, idx_v, *rest):
        bufs = rest[:NSLOT]
        gsems = rest[NSLOT : 2 * NSLOT]
        wsems = rest[2 * NSLOT : 3 * NSLOT]
        wid = lax.axis_index("s") * NC + lax.axis_index("c")
        base = wid * b_per_w
        pltpu.sync_copy(idx_hbm.at[pl.ds(base, b_per_w)], idx_v)

        def gather_desc(slot, c, blk):
            idx_c = idx_v.at[pl.ds(c * CHUNK, CHUNK)]
            if blk == 2:
                src = tail_hbm.at[idx_c]
            else:
                src = table_hbm.at[idx_c, pl.ds(blk * 128, 128)]
            return pltpu.make_async_copy(src, bufs[slot], gsems[slot])

        def write_desc(slot, c, blk):
            dst = out_hbm.at[pl.ds(base + c * CHUNK, CHUNK), pl.ds(blk * 128, 128)]
            return pltpu.make_async_copy(bufs[slot], dst, wsems[slot])

        def fire_gather(slot, c, blk):
            gather_desc(slot, c, blk).start()

        def fire_write(slot, c, blk):
            write_desc(slot, c, blk).start()

        for p in range(2):
            for blk in range(3):
                fire_gather(p * 3 + blk, p, blk)

        def body(t, carry):
            c2 = 2 * t
            for p in range(2):
                for blk in range(3):
                    slot = p * 3 + blk
                    gather_desc(slot, 0, blk).wait()
                    fire_write(slot, c2 + p, blk)
            for p in range(2):
                for blk in range(3):
                    slot = p * 3 + blk
                    write_desc(slot, 0, blk).wait()

                    @pl.when(c2 + p + 2 < n_chunks)
                    def _(slot=slot, p=p, blk=blk):
                        fire_gather(slot, c2 + p + 2, blk)

            return carry

        lax.fori_loop(0, n_chunks // 2, body, 0)

    return k


def kernel(inputs, table):
    bsz, seq = inputs.shape
    B = bsz * seq
    idx = inputs.reshape(B).astype(jnp.int32)
    tail = jnp.pad(table[:, 256:], ((0, 0), (0, 84)))
    info = plsc.get_sparse_core_info()
    NC, NS = info.num_cores, info.num_subcores
    NW = NC * NS
    b_per_w = B // NW
    n_chunks = b_per_w // CHUNK
    out = _make_gather(B, b_per_w, n_chunks, NC)(table, tail, idx)
    return out[:, :DIM].reshape(bsz, seq, DIM)
